# speedup vs baseline: 1.5652x; 1.5652x over previous
import jax
import jax.numpy as jnp
from jax.experimental import pallas as pl

T = 8192
D = 2048
E = 64
TOPK = 2
TILE_T = 512


def _router_kernel(x_ref, w_ref, bias_ref, scale_ref, w_out_ref, i_out_ref):
    acc = jnp.dot(x_ref[...], w_ref[...], preferred_element_type=jnp.float32)
    gate = acc[:, :E]
    up = acc[:, E:]
    s = jnp.abs(up * gate * jax.nn.sigmoid(gate))
    m = jnp.max(s, axis=1, keepdims=True)
    ex = jnp.exp(s - m)
    sm = ex / jnp.sum(ex, axis=1, keepdims=True)

    bias = bias_ref[0, :]
    scale = scale_ref[0, :]
    biased = sm + bias[None, :]
    lane = jax.lax.broadcasted_iota(jnp.int32, biased.shape, 1)

    m1 = jnp.max(biased, axis=1, keepdims=True)
    i1 = jnp.min(jnp.where(biased == m1, lane, E), axis=1, keepdims=True)
    mask1 = lane == i1
    rest = jnp.where(mask1, -jnp.inf, biased)
    m2 = jnp.max(rest, axis=1, keepdims=True)
    i2 = jnp.min(jnp.where(rest == m2, lane, E), axis=1, keepdims=True)
    mask2 = lane == i2

    w = 1.0 + sm * scale[None, :]
    w1 = jnp.sum(jnp.where(mask1, w, 0.0), axis=1, keepdims=True)
    w2 = jnp.sum(jnp.where(mask2, w, 0.0), axis=1, keepdims=True)

    w_out_ref[...] = jnp.concatenate([w1, w2], axis=1)
    i_out_ref[...] = jnp.concatenate([i1, i2], axis=1)


@jax.jit
def kernel(x, W_gate, W_up, extra_scale, extra_bias):
    W = jnp.concatenate([W_gate, W_up], axis=1)
    bias2d = extra_bias.reshape(1, E)
    scale2d = extra_scale.reshape(1, E)
    grid = (T // TILE_T,)
    weights, indices = pl.pallas_call(
        _router_kernel,
        grid=grid,
        in_specs=[
            pl.BlockSpec((TILE_T, D), lambda i: (i, 0)),
            pl.BlockSpec((D, 2 * E), lambda i: (0, 0)),
            pl.BlockSpec((1, E), lambda i: (0, 0)),
            pl.BlockSpec((1, E), lambda i: (0, 0)),
        ],
        out_specs=[
            pl.BlockSpec((TILE_T, TOPK), lambda i: (i, 0)),
            pl.BlockSpec((TILE_T, TOPK), lambda i: (i, 0)),
        ],
        out_shape=[
            jax.ShapeDtypeStruct((T, TOPK), jnp.float32),
            jax.ShapeDtypeStruct((T, TOPK), jnp.int32),
        ],
    )(x, W, bias2d, scale2d)
    return weights, indices


# speedup vs baseline: 3.2354x; 2.0671x over previous
import jax
import jax.numpy as jnp
from jax.experimental import pallas as pl

T = 8192
D = 2048
E = 64
TOPK = 2
TILE_T = 512


def _router_kernel(x_ref, w_ref, bias_ref, scale_ref, w_out_ref, i_out_ref):
    acc = jax.lax.dot_general(
        w_ref[...], x_ref[...],
        dimension_numbers=(((0,), (1,)), ((), ())),
        preferred_element_type=jnp.float32,
    )
    gate = acc[:E, :]
    up = acc[E:, :]
    s = jnp.abs(up * gate * jax.nn.sigmoid(gate))
    ex = jnp.exp(jnp.minimum(s, 80.0))
    sm = ex / jnp.sum(ex, axis=0, keepdims=True)

    biased = sm + bias_ref[...]
    row = jax.lax.broadcasted_iota(jnp.int32, biased.shape, 0)

    m1 = jnp.max(biased, axis=0, keepdims=True)
    i1 = jnp.min(jnp.where(biased == m1, row, E), axis=0, keepdims=True)
    mask1 = row == i1
    rest = jnp.where(mask1, -jnp.inf, biased)
    m2 = jnp.max(rest, axis=0, keepdims=True)
    i2 = jnp.min(jnp.where(rest == m2, row, E), axis=0, keepdims=True)
    mask2 = row == i2

    w = 1.0 + sm * scale_ref[...]
    w1 = jnp.sum(jnp.where(mask1, w, 0.0), axis=0, keepdims=True)
    w2 = jnp.sum(jnp.where(mask2, w, 0.0), axis=0, keepdims=True)

    w_out_ref[...] = jnp.concatenate([w1, w2], axis=0)
    i_out_ref[...] = jnp.concatenate([i1, i2], axis=0)


@jax.jit
def kernel(x, W_gate, W_up, extra_scale, extra_bias):
    W = jnp.concatenate([W_gate, W_up], axis=1)
    bias2d = extra_bias.reshape(E, 1)
    scale2d = extra_scale.reshape(E, 1)
    grid = (T // TILE_T,)
    w_t, i_t = pl.pallas_call(
        _router_kernel,
        grid=grid,
        in_specs=[
            pl.BlockSpec((TILE_T, D), lambda i: (i, 0)),
            pl.BlockSpec((D, 2 * E), lambda i: (0, 0)),
            pl.BlockSpec((E, 1), lambda i: (0, 0)),
            pl.BlockSpec((E, 1), lambda i: (0, 0)),
        ],
        out_specs=[
            pl.BlockSpec((TOPK, TILE_T), lambda i: (0, i)),
            pl.BlockSpec((TOPK, TILE_T), lambda i: (0, i)),
        ],
        out_shape=[
            jax.ShapeDtypeStruct((TOPK, T), jnp.float32),
            jax.ShapeDtypeStruct((TOPK, T), jnp.int32),
        ],
    )(x, W, bias2d, scale2d)
    return w_t.T, i_t.T
